# baseline (speedup 1.0000x reference)
import jax
import jax.numpy as jnp
from jax import lax
from jax.experimental import pallas as pl
from jax.experimental.pallas import tpu as pltpu
from jax.experimental.pallas import tpu_sc as plsc

EMB = 64
WIDE = 128
NEXP = 64
NW = 32
GSUB = 128
NG = 2
CHUNK = NG * GSUB


def _gather_body(table_hbm, idx_hbm, out_hbm,
                 idx_all, rows_a, rows_b,
                 gsem_a, gsem_b, wsem_a, wsem_b):
    tok = out_hbm.shape[0]
    per_w = tok // NW
    nch = per_w // CHUNK
    npairs = nch // 2
    nirows = per_w // GSUB
    wid = lax.axis_index("s") * 2 + lax.axis_index("c")
    row0 = wid * per_w
    irow0 = row0 // GSUB

    pltpu.sync_copy(idx_hbm.at[pl.ds(pl.multiple_of(irow0, 8), nirows)],
                    idx_all)

    def fire(c, rows_v, gsem):
        for j in range(NG):
            pltpu.async_copy(table_hbm.at[idx_all.at[c * NG + j]],
                             rows_v.at[pl.ds(j * GSUB, GSUB)], gsem)

    def wait_g(rows_v, gsem):
        for j in range(NG):
            pltpu.make_async_copy(table_hbm.at[idx_all.at[j]],
                                  rows_v.at[pl.ds(j * GSUB, GSUB)],
                                  gsem).wait()

    def fire_wb(c, rows_v, wsem):
        pltpu.async_copy(rows_v, out_hbm.at[pl.ds(row0 + c * CHUNK, CHUNK)],
                         wsem)

    def wait_wb(rows_v, wsem):
        pltpu.make_async_copy(rows_v, out_hbm.at[pl.ds(row0, CHUNK)],
                              wsem).wait()

    fire(0, rows_a, gsem_a)
    fire(1, rows_b, gsem_b)

    def body(i, _):
        c = 2 * i
        wait_g(rows_a, gsem_a)
        fire_wb(c, rows_a, wsem_a)
        wait_g(rows_b, gsem_b)
        fire_wb(c + 1, rows_b, wsem_b)
        wait_wb(rows_a, wsem_a)
        fire(c + 2, rows_a, gsem_a)
        wait_wb(rows_b, wsem_b)
        fire(c + 3, rows_b, gsem_b)
        return 0

    lax.fori_loop(0, npairs - 1, body, 0)

    wait_g(rows_a, gsem_a)
    fire_wb(nch - 2, rows_a, wsem_a)
    wait_g(rows_b, gsem_b)
    fire_wb(nch - 1, rows_b, wsem_b)
    wait_wb(rows_a, wsem_a)
    wait_wb(rows_b, wsem_b)


def _sc_gather(table2, idx2d, tok):
    mesh = plsc.VectorSubcoreMesh(core_axis_name="c", subcore_axis_name="s")
    nirows = tok // NW // GSUB
    return pl.kernel(
        _gather_body,
        out_type=jax.ShapeDtypeStruct((tok, WIDE), jnp.float32),
        mesh=mesh,
        scratch_types=[
            pltpu.VMEM((nirows, GSUB), jnp.int32),
            pltpu.VMEM((CHUNK, WIDE), jnp.float32),
            pltpu.VMEM((CHUNK, WIDE), jnp.float32),
            pltpu.SemaphoreType.DMA,
            pltpu.SemaphoreType.DMA,
            pltpu.SemaphoreType.DMA,
            pltpu.SemaphoreType.DMA,
        ],
    )(table2, idx2d)


def _router_body(h2_ref, xp_ref, w_ref, h_ref, out_ref):
    h2 = h2_ref[...]
    odd = (xp_ref[...] & 1) == 1
    h = jnp.where(odd, h2[:, EMB:], h2[:, :EMB])
    h_ref[...] = h
    out_ref[...] = lax.dot_general(
        h, w_ref[...], (((1,), (1,)), ((), ())),
        preferred_element_type=jnp.float32)


def _router(h2, xcol, router_w, tok, blk=2048):
    return pl.pallas_call(
        _router_body,
        grid=(tok // blk,),
        in_specs=[
            pl.BlockSpec((blk, WIDE), lambda i: (i, 0)),
            pl.BlockSpec((blk, 1), lambda i: (i, 0)),
            pl.BlockSpec((NEXP, EMB), lambda i: (0, 0)),
        ],
        out_specs=[
            pl.BlockSpec((blk, EMB), lambda i: (i, 0)),
            pl.BlockSpec((blk, NEXP), lambda i: (i, 0)),
        ],
        out_shape=[
            jax.ShapeDtypeStruct((tok, EMB), jnp.float32),
            jax.ShapeDtypeStruct((tok, NEXP), jnp.float32),
        ],
    )(h2, xcol, router_w)


def kernel(x, embed_table, router_w):
    b, l = x.shape
    tok = b * l
    vocab = embed_table.shape[0]
    xi = x.astype(jnp.int32).reshape(tok)
    idx2d = (xi >> 1).reshape(tok // GSUB, GSUB)
    table2 = embed_table.reshape(vocab // 2, WIDE)
    h2 = _sc_gather(table2, idx2d, tok)
    h_flat, logits_flat = _router(h2, xi.reshape(tok, 1), router_w, tok)
    return h_flat.reshape(b, l, EMB), logits_flat.reshape(b, l, NEXP)
